# baseline (speedup 1.0000x reference)
import functools

import jax
import jax.numpy as jnp
from jax.experimental import pallas as pl
from jax.experimental.pallas import tpu as pltpu

E = 8
K = 2
LB_COEFF = 0.01
BLK = 512


def _routing_body(x_ref, gw_ref, i1_ref, i2_ref, p1_ref, p2_ref, psum_ref):
    x = x_ref[...]
    gw = gw_ref[...]
    scores = jnp.dot(x, gw, preferred_element_type=jnp.float32)
    lane = jax.lax.broadcasted_iota(jnp.int32, scores.shape, 1)
    neg_inf = jnp.float32(-jnp.inf)
    scores = jnp.where(lane < E, scores, neg_inf)
    probs = jax.nn.softmax(scores, axis=-1)
    m1 = jnp.max(probs, axis=1, keepdims=True)
    i1 = jnp.min(jnp.where(probs == m1, lane, 128), axis=1, keepdims=True)
    probs2 = jnp.where(lane == i1, jnp.float32(-1.0), probs)
    m2 = jnp.max(probs2, axis=1, keepdims=True)
    i2 = jnp.min(jnp.where(probs2 == m2, lane, 128), axis=1, keepdims=True)
    e2 = jnp.exp(m2 - m1)
    denom = 1.0 + e2
    p1 = 1.0 / denom
    p2 = e2 / denom
    i1_ref[...] = i1
    i2_ref[...] = i2
    p1_ref[...] = p1
    p2_ref[...] = p2
    psum_ref[...] = jnp.sum(probs, axis=0)


def _ffn_body(be_ref, act_ref, x_ref, w1_ref, b1_ref, w2_ref, b2_ref, y_ref):
    g = pl.program_id(0)

    @pl.when(act_ref[g] > 0)
    def _():
        h = jnp.dot(x_ref[...], w1_ref[0], preferred_element_type=jnp.float32)
        h = jnp.maximum(h + b1_ref[0], 0.0)
        y = jnp.dot(h, w2_ref[0], preferred_element_type=jnp.float32)
        y_ref[...] = y + b2_ref[0]


def kernel(x, gate_w, W1, b1, W2, b2):
    b, s, h = x.shape
    hd = W1.shape[-1]
    x_flat = x.reshape(-1, h)
    n = x_flat.shape[0]

    gw_pad = jnp.zeros((h, 128), jnp.float32).at[:, :E].set(gate_w)
    out_shapes = (
        jax.ShapeDtypeStruct((n, 1), jnp.int32),
        jax.ShapeDtypeStruct((n, 1), jnp.int32),
        jax.ShapeDtypeStruct((n, 1), jnp.float32),
        jax.ShapeDtypeStruct((n, 1), jnp.float32),
        jax.ShapeDtypeStruct((128,), jnp.float32),
    )
    i1b, i2b, p1b, p2b, psum = pl.pallas_call(
        _routing_body,
        out_shape=out_shapes,
    )(x_flat, gw_pad)
    i1, i2 = i1b[:, 0], i2b[:, 0]
    p1, p2 = p1b[:, 0], p2b[:, 0]

    e_all = jnp.concatenate([i1, i2])
    onehot = jax.nn.one_hot(e_all, E, dtype=jnp.int32)
    ranks_all = jnp.cumsum(onehot, axis=0) - onehot
    rank = jnp.sum(ranks_all * onehot, axis=1)
    counts = jnp.sum(onehot, axis=0)
    blocks_per_e = (counts + BLK - 1) // BLK
    cum_blocks = jnp.cumsum(blocks_per_e)
    pad_off = (cum_blocks - blocks_per_e) * BLK
    dst = jnp.sum(pad_off[None, :] * onehot, axis=1) + rank

    G = (K * n) // BLK + E
    R = G * BLK
    total_active = cum_blocks[E - 1]
    g_ids = jnp.arange(G, dtype=jnp.int32)
    g_clamped = jnp.minimum(g_ids, total_active - 1)
    block_expert = jnp.searchsorted(cum_blocks, g_clamped, side="right").astype(jnp.int32)
    active = (g_ids < total_active).astype(jnp.int32)

    tok_all = jnp.concatenate([jnp.arange(n, dtype=jnp.int32)] * K)
    row_tok = jnp.zeros((R,), jnp.int32).at[dst].set(tok_all)

    x_sorted = jnp.take(x_flat, row_tok, axis=0)

    grid_spec = pltpu.PrefetchScalarGridSpec(
        num_scalar_prefetch=2,
        grid=(G,),
        in_specs=[
            pl.BlockSpec((BLK, h), lambda g, be, act: (g, 0)),
            pl.BlockSpec((1, h, hd), lambda g, be, act: (be[g], 0, 0)),
            pl.BlockSpec((1, 1, hd), lambda g, be, act: (be[g], 0, 0)),
            pl.BlockSpec((1, hd, h), lambda g, be, act: (be[g], 0, 0)),
            pl.BlockSpec((1, 1, h), lambda g, be, act: (be[g], 0, 0)),
        ],
        out_specs=pl.BlockSpec((BLK, h), lambda g, be, act: (g, 0)),
    )
    y = pl.pallas_call(
        _ffn_body,
        grid_spec=grid_spec,
        out_shape=jax.ShapeDtypeStruct((R, h), jnp.float32),
        compiler_params=pltpu.CompilerParams(
            dimension_semantics=("arbitrary",),
        ),
    )(block_expert, active, x_sorted, W1, b1[:, None, :], W2, b2[:, None, :])

    pos1, pos2 = dst[:n], dst[n:]
    out = p1[:, None] * jnp.take(y, pos1, axis=0) + p2[:, None] * jnp.take(y, pos2, axis=0)
    out = out.reshape(b, s, h)

    expert_usage = psum[:E] / n
    log_input = jax.nn.log_softmax(expert_usage, axis=0)
    uniform = jnp.ones_like(expert_usage) / E
    kl = jnp.sum(uniform * (jnp.log(uniform) - log_input)) / E
    load_balance_loss = LB_COEFF * kl
    return out, load_balance_loss


# speedup vs baseline: 1.0072x; 1.0072x over previous
import functools

import jax
import jax.numpy as jnp
from jax.experimental import pallas as pl
from jax.experimental.pallas import tpu as pltpu

E = 8
K = 2
LB_COEFF = 0.01
BLK = 256


def _routing_body(x_ref, gw_ref, i1_ref, i2_ref, p1_ref, p2_ref, psum_ref):
    x = x_ref[...]
    gw = gw_ref[...]
    scores = jnp.dot(x, gw, preferred_element_type=jnp.float32)
    lane = jax.lax.broadcasted_iota(jnp.int32, scores.shape, 1)
    neg_inf = jnp.float32(-jnp.inf)
    scores = jnp.where(lane < E, scores, neg_inf)
    probs = jax.nn.softmax(scores, axis=-1)
    m1 = jnp.max(probs, axis=1, keepdims=True)
    i1 = jnp.min(jnp.where(probs == m1, lane, 128), axis=1, keepdims=True)
    probs2 = jnp.where(lane == i1, jnp.float32(-1.0), probs)
    m2 = jnp.max(probs2, axis=1, keepdims=True)
    i2 = jnp.min(jnp.where(probs2 == m2, lane, 128), axis=1, keepdims=True)
    e2 = jnp.exp(m2 - m1)
    denom = 1.0 + e2
    p1 = 1.0 / denom
    p2 = e2 / denom
    i1_ref[...] = i1
    i2_ref[...] = i2
    p1_ref[...] = p1
    p2_ref[...] = p2
    psum_ref[...] = jnp.sum(probs, axis=0)


def _ffn_body(be_ref, act_ref, x_ref, w1_ref, b1_ref, w2_ref, b2_ref, y_ref):
    g = pl.program_id(0)

    @pl.when(act_ref[g] > 0)
    def _():
        h = jnp.dot(x_ref[...], w1_ref[0], preferred_element_type=jnp.float32)
        h = jnp.maximum(h + b1_ref[0], 0.0)
        y = jnp.dot(h, w2_ref[0], preferred_element_type=jnp.float32)
        y_ref[...] = y + b2_ref[0]


def kernel(x, gate_w, W1, b1, W2, b2):
    b, s, h = x.shape
    hd = W1.shape[-1]
    x_flat = x.reshape(-1, h)
    n = x_flat.shape[0]

    gw_pad = jnp.zeros((h, 128), jnp.float32).at[:, :E].set(gate_w)
    out_shapes = (
        jax.ShapeDtypeStruct((n, 1), jnp.int32),
        jax.ShapeDtypeStruct((n, 1), jnp.int32),
        jax.ShapeDtypeStruct((n, 1), jnp.float32),
        jax.ShapeDtypeStruct((n, 1), jnp.float32),
        jax.ShapeDtypeStruct((128,), jnp.float32),
    )
    i1b, i2b, p1b, p2b, psum = pl.pallas_call(
        _routing_body,
        out_shape=out_shapes,
    )(x_flat, gw_pad)
    i1, i2 = i1b[:, 0], i2b[:, 0]
    p1, p2 = p1b[:, 0], p2b[:, 0]

    e_all = jnp.concatenate([i1, i2])
    onehot = jax.nn.one_hot(e_all, E, dtype=jnp.int32)
    ranks_all = jnp.cumsum(onehot, axis=0) - onehot
    rank = jnp.sum(ranks_all * onehot, axis=1)
    counts = jnp.sum(onehot, axis=0)
    blocks_per_e = (counts + BLK - 1) // BLK
    cum_blocks = jnp.cumsum(blocks_per_e)
    pad_off = (cum_blocks - blocks_per_e) * BLK
    dst = jnp.sum(pad_off[None, :] * onehot, axis=1) + rank

    G = (K * n) // BLK + E
    R = G * BLK
    total_active = cum_blocks[E - 1]
    g_ids = jnp.arange(G, dtype=jnp.int32)
    g_clamped = jnp.minimum(g_ids, total_active - 1)
    block_expert = jnp.searchsorted(cum_blocks, g_clamped, side="right").astype(jnp.int32)
    active = (g_ids < total_active).astype(jnp.int32)

    tok_all = jnp.concatenate([jnp.arange(n, dtype=jnp.int32)] * K)
    row_tok = jnp.zeros((R,), jnp.int32).at[dst].set(tok_all)

    x_sorted = jnp.take(x_flat, row_tok, axis=0)

    grid_spec = pltpu.PrefetchScalarGridSpec(
        num_scalar_prefetch=2,
        grid=(G,),
        in_specs=[
            pl.BlockSpec((BLK, h), lambda g, be, act: (g, 0)),
            pl.BlockSpec((1, h, hd), lambda g, be, act: (be[g], 0, 0)),
            pl.BlockSpec((1, 1, hd), lambda g, be, act: (be[g], 0, 0)),
            pl.BlockSpec((1, hd, h), lambda g, be, act: (be[g], 0, 0)),
            pl.BlockSpec((1, 1, h), lambda g, be, act: (be[g], 0, 0)),
        ],
        out_specs=pl.BlockSpec((BLK, h), lambda g, be, act: (g, 0)),
    )
    y = pl.pallas_call(
        _ffn_body,
        grid_spec=grid_spec,
        out_shape=jax.ShapeDtypeStruct((R, h), jnp.float32),
        compiler_params=pltpu.CompilerParams(
            dimension_semantics=("arbitrary",),
            vmem_limit_bytes=120 * 1024 * 1024,
        ),
    )(block_expert, active, x_sorted, W1, b1[:, None, :], W2, b2[:, None, :])

    pos1, pos2 = dst[:n], dst[n:]
    out = p1[:, None] * jnp.take(y, pos1, axis=0) + p2[:, None] * jnp.take(y, pos2, axis=0)
    out = out.reshape(b, s, h)

    expert_usage = psum[:E] / n
    log_input = jax.nn.log_softmax(expert_usage, axis=0)
    uniform = jnp.ones_like(expert_usage) / E
    kl = jnp.sum(uniform * (jnp.log(uniform) - log_input)) / E
    load_balance_loss = LB_COEFF * kl
    return out, load_balance_loss
